# Initial kernel scaffold; baseline (speedup 1.0000x reference)
import jax
import jax.numpy as jnp
from jax.experimental import pallas as pl


def kernel(x, n_id, edge_index, W1, b1, W2, b2, W3, b3, L1, bl1, L2, bl2, L3, bl3):
    raise NotImplementedError("write your pallas kernel here")


# speedup vs baseline: 4.7089x; 4.7089x over previous
import jax
import jax.numpy as jnp
from jax.experimental import pallas as pl
from jax.experimental.pallas import tpu as pltpu

_B, _N, _E = 32, 77, 1232
_C0, _H1, _H2, _H3 = 128, 64, 256, 512
_HID, _MID, _OUT = 1024, 218, 10


def _fused_body(n_id_ref, ei_ref, x_hbm, W1, b1, W2, b2, W3, b3,
                L1blk, bl1, L2, bl2, L3, bl3, out_ref,
                xt, h3, acc, sem):
    j = pl.program_id(0)

    @pl.when(j == 0)
    def _prologue():
        for i in range(_N):
            pltpu.make_async_copy(
                x_hbm.at[:, pl.ds(n_id_ref[i], 1), :],
                xt.at[:, pl.ds(i, 1), :], sem).start()
        for i in range(_N):
            pltpu.make_async_copy(
                x_hbm.at[:, pl.ds(n_id_ref[i], 1), :],
                xt.at[:, pl.ds(i, 1), :], sem).wait()

        src = ei_ref[0:1, :]
        dst = ei_ref[1:2, :]
        ion = jax.lax.broadcasted_iota(jnp.int32, (_N, _E), 0)
        S = (ion == src).astype(jnp.float32)
        D = (ion == dst).astype(jnp.float32)
        A = jax.lax.dot_general(D, S, (((1,), (1,)), ((), ())),
                                preferred_element_type=jnp.float32)
        cnt = jnp.sum(A, axis=1, keepdims=True)
        An = A / jnp.maximum(cnt, 1.0)

        def conv(h, Wr, br, cin):
            ag = jnp.dot(An, h, preferred_element_type=jnp.float32)
            o = (jnp.dot(h, Wr[:cin, :], preferred_element_type=jnp.float32)
                 + jnp.dot(ag, Wr[cin:, :], preferred_element_type=jnp.float32)
                 + br[...])
            nrm = jnp.sqrt(jnp.sum(o * o, axis=-1, keepdims=True))
            o = o / jnp.maximum(nrm, 1e-12)
            return jnp.maximum(o, 0.0)

        def batch_body(b, _):
            h0 = xt[pl.ds(b, 1), :, :].reshape(_N, _C0)
            h1 = conv(h0, W1, b1, _C0)
            h2 = conv(h1, W2, b2, _H1)
            hb = conv(h2, W3, b3, _H2)
            h3[pl.ds(b, 1), :, :] = hb.reshape(1, _N, _H3)
            return 0

        jax.lax.fori_loop(0, _B, batch_body, 0)

    hj = h3[:, pl.ds(j, 1), :].reshape(_B, _H3)
    prod = jnp.dot(hj, L1blk[0], preferred_element_type=jnp.float32)

    @pl.when(j == 0)
    def _init():
        acc[...] = prod

    @pl.when(j > 0)
    def _accum():
        acc[...] += prod

    @pl.when(j == _N - 1)
    def _epilogue():
        y = jnp.maximum(acc[...] + bl1[...], 0.0)
        y = jnp.maximum(jnp.dot(y, L2[...], preferred_element_type=jnp.float32)
                        + bl2[...], 0.0)
        out_ref[...] = (jnp.dot(y, L3[...], preferred_element_type=jnp.float32)
                        + bl3[...])


def kernel(x, n_id, edge_index, W1, b1, W2, b2, W3, b3,
           L1, bl1, L2, bl2, L3, bl3):
    n_id = n_id.astype(jnp.int32)
    edge_index = edge_index.astype(jnp.int32)
    L1r = L1.reshape(_N, _H3, _HID)

    vmem = pl.BlockSpec(memory_space=pltpu.VMEM)
    out = pl.pallas_call(
        _fused_body,
        grid=(_N,),
        in_specs=[
            pl.BlockSpec(memory_space=pltpu.SMEM),
            vmem,
            pl.BlockSpec(memory_space=pl.ANY),
            vmem, vmem, vmem, vmem, vmem, vmem,
            pl.BlockSpec((1, _H3, _HID), lambda j: (j, 0, 0)),
            vmem, vmem, vmem, vmem, vmem,
        ],
        out_specs=pl.BlockSpec((_B, _OUT), lambda j: (0, 0)),
        out_shape=jax.ShapeDtypeStruct((_B, _OUT), jnp.float32),
        scratch_shapes=[
            pltpu.VMEM((_B, _N, _C0), jnp.float32),
            pltpu.VMEM((_B, _N, _H3), jnp.float32),
            pltpu.VMEM((_B, _HID), jnp.float32),
            pltpu.SemaphoreType.DMA,
        ],
    )(n_id, edge_index, x,
      W1, b1.reshape(1, -1), W2, b2.reshape(1, -1), W3, b3.reshape(1, -1),
      L1r, bl1.reshape(1, -1), L2, bl2.reshape(1, -1), L3, bl3.reshape(1, -1))
    return out


# speedup vs baseline: 6.0671x; 1.2885x over previous
import jax
import jax.numpy as jnp
from jax.experimental import pallas as pl
from jax.experimental.pallas import tpu as pltpu

_B, _N, _E = 32, 77, 1232
_C0, _H1, _H2, _H3 = 128, 64, 256, 512
_HID, _MID, _OUT = 1024, 218, 10
_NPB = 7
_GRID = _N // _NPB


def _fused_body(n_id_ref, ei_ref, x_hbm, W1, b1, W2, b2, W3, b3,
                L1blk, bl1, L2, bl2, L3, bl3, out_ref,
                xt, h3, acc, sem):
    j = pl.program_id(0)

    @pl.when(j == 0)
    def _prologue():
        for i in range(_N):
            pltpu.make_async_copy(
                x_hbm.at[:, pl.ds(n_id_ref[i], 1), :],
                xt.at[:, pl.ds(i, 1), :], sem).start()
        for i in range(_N):
            pltpu.make_async_copy(
                x_hbm.at[:, pl.ds(n_id_ref[i], 1), :],
                xt.at[:, pl.ds(i, 1), :], sem).wait()

        src = ei_ref[0:1, :]
        dst = ei_ref[1:2, :]
        ion = jax.lax.broadcasted_iota(jnp.int32, (_N, _E), 0)
        S = (ion == src).astype(jnp.float32)
        D = (ion == dst).astype(jnp.float32)
        A = jax.lax.dot_general(D, S, (((1,), (1,)), ((), ())),
                                preferred_element_type=jnp.float32)
        cnt = jnp.sum(A, axis=1, keepdims=True)
        An = A / jnp.maximum(cnt, 1.0)

        def conv(h, Wr, br, cin):
            ag = jnp.dot(An, h, preferred_element_type=jnp.float32)
            o = (jnp.dot(h, Wr[:cin, :], preferred_element_type=jnp.float32)
                 + jnp.dot(ag, Wr[cin:, :], preferred_element_type=jnp.float32)
                 + br[...])
            nrm = jnp.sqrt(jnp.sum(o * o, axis=-1, keepdims=True))
            o = o / jnp.maximum(nrm, 1e-12)
            return jnp.maximum(o, 0.0)

        def batch_body(b, _):
            h0 = xt[pl.ds(b, 1), :, :].reshape(_N, _C0)
            h1 = conv(h0, W1, b1, _C0)
            h2 = conv(h1, W2, b2, _H1)
            hb = conv(h2, W3, b3, _H2)
            h3[pl.ds(b, 1), :, :] = hb.reshape(1, _N, _H3)
            return 0

        jax.lax.fori_loop(0, _B, batch_body, 0)

    prod = jnp.dot(h3[:, pl.ds(j * _NPB, 1), :].reshape(_B, _H3), L1blk[0],
                   preferred_element_type=jnp.float32)
    for i in range(1, _NPB):
        prod += jnp.dot(h3[:, pl.ds(j * _NPB + i, 1), :].reshape(_B, _H3),
                        L1blk[i], preferred_element_type=jnp.float32)

    @pl.when(j == 0)
    def _init():
        acc[...] = prod

    @pl.when(j > 0)
    def _accum():
        acc[...] += prod

    @pl.when(j == _GRID - 1)
    def _epilogue():
        y = jnp.maximum(acc[...] + bl1[...], 0.0)
        y = jnp.maximum(jnp.dot(y, L2[...], preferred_element_type=jnp.float32)
                        + bl2[...], 0.0)
        out_ref[...] = (jnp.dot(y, L3[...], preferred_element_type=jnp.float32)
                        + bl3[...])


def kernel(x, n_id, edge_index, W1, b1, W2, b2, W3, b3,
           L1, bl1, L2, bl2, L3, bl3):
    n_id = n_id.astype(jnp.int32)
    edge_index = edge_index.astype(jnp.int32)
    L1r = L1.reshape(_N, _H3, _HID)

    vmem = pl.BlockSpec(memory_space=pltpu.VMEM)
    out = pl.pallas_call(
        _fused_body,
        grid=(_GRID,),
        in_specs=[
            pl.BlockSpec(memory_space=pltpu.SMEM),
            vmem,
            pl.BlockSpec(memory_space=pl.ANY),
            vmem, vmem, vmem, vmem, vmem, vmem,
            pl.BlockSpec((_NPB, _H3, _HID), lambda j: (j, 0, 0)),
            vmem, vmem, vmem, vmem, vmem,
        ],
        out_specs=pl.BlockSpec((_B, _OUT), lambda j: (0, 0)),
        out_shape=jax.ShapeDtypeStruct((_B, _OUT), jnp.float32),
        scratch_shapes=[
            pltpu.VMEM((_B, _N, _C0), jnp.float32),
            pltpu.VMEM((_B, _N, _H3), jnp.float32),
            pltpu.VMEM((_B, _HID), jnp.float32),
            pltpu.SemaphoreType.DMA,
        ],
    )(n_id, edge_index, x,
      W1, b1.reshape(1, -1), W2, b2.reshape(1, -1), W3, b3.reshape(1, -1),
      L1r, bl1.reshape(1, -1), L2, bl2.reshape(1, -1), L3, bl3.reshape(1, -1))
    return out
